# baseline (speedup 1.0000x reference)
import functools

import jax
import jax.numpy as jnp
from jax import lax
from jax.experimental import pallas as pl
from jax.experimental.pallas import tpu as pltpu
from jax.experimental.pallas import tpu_sc as plsc

VOCAB = 1000000
EMBED = 32
BATCH = 16384
FIELDS = 26
_NC, _NS = 2, 16
_NW = _NC * _NS

_NBLK_FULL = VOCAB // 128
_TAIL = VOCAB - _NBLK_FULL * 128

_mesh = plsc.VectorSubcoreMesh(core_axis_name="c", subcore_axis_name="s")


def _wid():
    return lax.axis_index("s") * _NC + lax.axis_index("c")


def _iota16():
    return lax.iota(jnp.int32, 16)


@functools.partial(
    pl.kernel,
    mesh=_mesh,
    compiler_params=pltpu.CompilerParams(use_tc_tiling_on_sc=True,
                                         needs_layout_passes=False),
    out_type=jax.ShapeDtypeStruct((VOCAB // 4, 128), jnp.float32),
    scratch_types=[
        pltpu.VMEM((2, 32, 128), jnp.float32),
        pltpu.VMEM((2, 32, 128), jnp.float32),
        pltpu.VMEM((32, 64), jnp.float32),
        pltpu.VMEM((16, 128), jnp.float32),
        pltpu.SemaphoreType.DMA((2,)),
        pltpu.SemaphoreType.DMA((2,)),
    ],
)
def _k1(wt_hbm, lin_hbm, src, dst, tsrc, tdst, isem, osem):
    w = _wid()
    nw = (_NBLK_FULL - 1 - w) // 32 + 1
    iota = _iota16()

    def start_in(k, s):
        j = w + 32 * k
        pltpu.async_copy(wt_hbm.at[:, pl.ds(j * 128, 128)], src.at[s],
                         isem.at[s])

    def wait_in(k, s):
        j = w + 32 * k
        pltpu.make_async_copy(wt_hbm.at[:, pl.ds(j * 128, 128)], src.at[s],
                              isem.at[s]).wait()

    def start_out(k, s):
        j = w + 32 * k
        pltpu.async_copy(dst.at[s], lin_hbm.at[pl.ds(32 * j, 32)], osem.at[s])

    def wait_out(k, s):
        j = w + 32 * k
        pltpu.make_async_copy(dst.at[s], lin_hbm.at[pl.ds(32 * j, 32)],
                              osem.at[s]).wait()

    def transpose_block(s):
        def body_r(wr, carry):
            for kk in range(8):
                rows = iota + 16 * (kk % 2)
                col = jnp.broadcast_to(4 * wr + kk // 2, (16,)).astype(jnp.int32)
                v = plsc.load_gather(src.at[s], [rows, col])
                dst[s, wr, pl.ds(16 * kk, 16)] = v
            return carry
        lax.fori_loop(0, 32, body_r, 0)

    start_in(0, 0)
    start_in(1, 1)

    def body_p(p, carry):
        for s in (0, 1):
            k = 2 * p + s

            @pl.when(k < nw)
            def _():
                wait_in(k, s)

                @pl.when(k >= 2)
                def _():
                    wait_out(k - 2, s)

                transpose_block(s)
                start_out(k, s)

                @pl.when(k + 2 < nw)
                def _():
                    start_in(k + 2, s)
        return carry

    lax.fori_loop(0, (245 + 1) // 2, body_p, 0)

    for s in (0, 1):
        k_last = nw - 1 - ((nw - 1 - s) % 2)
        wait_out(k_last, s)

    @pl.when(w == _NBLK_FULL % 32)
    def _():
        pltpu.sync_copy(wt_hbm.at[:, pl.ds(_NBLK_FULL * 128, _TAIL)], tsrc)

        def body_r(wr, carry):
            for kk in range(8):
                rows = iota + 16 * (kk % 2)
                col = jnp.broadcast_to(4 * wr + kk // 2, (16,)).astype(jnp.int32)
                v = plsc.load_gather(tsrc, [rows, col])
                tdst[wr, pl.ds(16 * kk, 16)] = v
            return carry
        lax.fori_loop(0, _TAIL // 4, body_r, 0)
        pltpu.sync_copy(tdst, lin_hbm.at[pl.ds(_NBLK_FULL * 32, _TAIL // 4)])


_NCOL = FIELDS * (BATCH // 128)
_COL_PER_W = _NCOL // _NW
_IDX_PER_W = _COL_PER_W * 128


@functools.partial(
    pl.kernel,
    mesh=_mesh,
    compiler_params=pltpu.CompilerParams(use_tc_tiling_on_sc=False,
                                         needs_layout_passes=False),
    out_type=jax.ShapeDtypeStruct((FIELDS * 4 * 128 * 8, 128), jnp.float32),
    scratch_types=[
        pltpu.VMEM((_IDX_PER_W,), jnp.int32),
        pltpu.VMEM((2, 128, 32), jnp.float32),
        pltpu.VMEM((2, 32, 128), jnp.float32),
        pltpu.SemaphoreType.DMA((2,)),
        pltpu.SemaphoreType.DMA((2,)),
    ],
)
def _k2(table_hbm, idx_hbm, out_hbm, idxv, rows, dest, gsem, osem):
    w = _wid()
    iota = _iota16()

    pltpu.sync_copy(idx_hbm.at[pl.ds(w * _IDX_PER_W, _IDX_PER_W)], idxv)

    def start_gather(i, s):
        pltpu.async_copy(table_hbm.at[idxv.at[pl.ds(128 * i, 128)]],
                         rows.at[s], gsem.at[s])

    def wait_gather(i, s):
        pltpu.make_async_copy(table_hbm.at[idxv.at[pl.ds(128 * i, 128)]],
                              rows.at[s], gsem.at[s]).wait()

    def out_bases(i):
        t = w * _COL_PER_W + i
        f = t // 128
        bb = t % 128
        return [4096 * f + 1024 * ii + 8 * bb for ii in range(4)]

    def start_outs(i, s):
        for ii, base in enumerate(out_bases(i)):
            pltpu.async_copy(dest.at[s].at[pl.ds(8 * ii, 8)],
                             out_hbm.at[pl.ds(base, 8)], osem.at[s])

    def wait_outs(i, s):
        for ii, base in enumerate(out_bases(i)):
            pltpu.make_async_copy(dest.at[s].at[pl.ds(8 * ii, 8)],
                                  out_hbm.at[pl.ds(base, 8)],
                                  osem.at[s]).wait()

    def transpose_col(s):
        def body_r(r, carry):
            col = jnp.broadcast_to(r, (16,)).astype(jnp.int32)
            for m in range(8):
                v = plsc.load_gather(rows.at[s], [iota + 16 * m, col])
                dest[s, r, pl.ds(16 * m, 16)] = v
            return carry
        lax.fori_loop(0, 32, body_r, 0)

    start_gather(0, 0)
    start_gather(1, 1)

    def body_p(p, carry):
        for s in (0, 1):
            i = 2 * p + s
            wait_gather(i, s)

            @pl.when(i >= 2)
            def _():
                wait_outs(i - 2, s)

            transpose_col(s)
            start_outs(i, s)

            @pl.when(i + 2 < _COL_PER_W)
            def _():
                start_gather(i + 2, s)
        return carry

    lax.fori_loop(0, _COL_PER_W // 2, body_p, 0)

    for s in (0, 1):
        wait_outs(_COL_PER_W - 2 + s, s)


def kernel(input, weight):
    idx_t = input.T.reshape(-1)
    if idx_t.dtype != jnp.int32:
        idx_t = idx_t.astype(jnp.int32)
    wt = weight.T
    lin = _k1(wt)
    table = lin.reshape(VOCAB, EMBED)
    out_flat = _k2(table, idx_t)
    o = out_flat.reshape(FIELDS, 4, 128, 8, 128)
    o = o.transpose(2, 4, 0, 1, 3)
    return o.reshape(BATCH, FIELDS, EMBED)


# speedup vs baseline: 2.8310x; 2.8310x over previous
import functools

import jax
import jax.numpy as jnp
from jax import lax
from jax.experimental import pallas as pl
from jax.experimental.pallas import tpu as pltpu
from jax.experimental.pallas import tpu_sc as plsc

VOCAB = 1000000
EMBED = 32
BATCH = 16384
FIELDS = 26
_NC, _NS = 2, 16
_NW = _NC * _NS

_NBLK_FULL = VOCAB // 128
_TAIL = VOCAB - _NBLK_FULL * 128

_mesh = plsc.VectorSubcoreMesh(core_axis_name="c", subcore_axis_name="s")


def _wid():
    return lax.axis_index("s") * _NC + lax.axis_index("c")


def _iota16():
    return lax.iota(jnp.int32, 16)


@functools.partial(
    pl.kernel,
    mesh=_mesh,
    compiler_params=pltpu.CompilerParams(use_tc_tiling_on_sc=True,
                                         needs_layout_passes=False),
    out_type=jax.ShapeDtypeStruct((VOCAB // 4, 128), jnp.float32),
    scratch_types=[
        pltpu.VMEM((2, 32, 128), jnp.float32),
        pltpu.VMEM((2, 32, 128), jnp.float32),
        pltpu.VMEM((32, 64), jnp.float32),
        pltpu.VMEM((16, 128), jnp.float32),
        pltpu.SemaphoreType.DMA((2,)),
        pltpu.SemaphoreType.DMA((2,)),
    ],
)
def _k1(wt_hbm, lin_hbm, src, dst, tsrc, tdst, isem, osem):
    w = _wid()
    nw = (_NBLK_FULL - 1 - w) // 32 + 1
    iota = _iota16()

    def start_in(k, s):
        j = w + 32 * k
        pltpu.async_copy(wt_hbm.at[:, pl.ds(j * 128, 128)], src.at[s],
                         isem.at[s])

    def wait_in(k, s):
        j = w + 32 * k
        pltpu.make_async_copy(wt_hbm.at[:, pl.ds(j * 128, 128)], src.at[s],
                              isem.at[s]).wait()

    def start_out(k, s):
        j = w + 32 * k
        pltpu.async_copy(dst.at[s], lin_hbm.at[pl.ds(32 * j, 32)], osem.at[s])

    def wait_out(k, s):
        j = w + 32 * k
        pltpu.make_async_copy(dst.at[s], lin_hbm.at[pl.ds(32 * j, 32)],
                              osem.at[s]).wait()

    def transpose_block(src_ref, dst_ref, nl0):
        cols = [iota + 16 * q for q in range(nl0)]
        wrows = [lax.shift_right_logical(c, 2) for c in cols]
        cbases = [lax.shift_left(c & 3, 5) for c in cols]

        def body(e0, em):
            for q in range(nl0):
                v = plsc.load_gather(src_ref, [em, cols[q]])
                plsc.store_scatter(dst_ref, [wrows[q], cbases[q] + em], v)
            return (em + 1) & 31

        lax.fori_loop(0, 32, body, iota)

    start_in(0, 0)
    start_in(1, 1)

    def body_p(p, carry):
        for s in (0, 1):
            k = 2 * p + s

            @pl.when(k < nw)
            def _():
                wait_in(k, s)

                @pl.when(k >= 2)
                def _():
                    wait_out(k - 2, s)

                transpose_block(src.at[s], dst.at[s], 8)
                start_out(k, s)

                @pl.when(k + 2 < nw)
                def _():
                    start_in(k + 2, s)
        return carry

    lax.fori_loop(0, (245 + 1) // 2, body_p, 0)

    for s in (0, 1):
        k_last = nw - 1 - ((nw - 1 - s) % 2)
        wait_out(k_last, s)

    @pl.when(w == _NBLK_FULL % 32)
    def _():
        pltpu.sync_copy(wt_hbm.at[:, pl.ds(_NBLK_FULL * 128, _TAIL)], tsrc)
        transpose_block(tsrc, tdst, _TAIL // 16)
        pltpu.sync_copy(tdst, lin_hbm.at[pl.ds(_NBLK_FULL * 32, _TAIL // 4)])


_NCOL = FIELDS * (BATCH // 128)
_COL_PER_W = _NCOL // _NW
_IDX_PER_W = _COL_PER_W * 128


@functools.partial(
    pl.kernel,
    mesh=_mesh,
    compiler_params=pltpu.CompilerParams(use_tc_tiling_on_sc=False,
                                         needs_layout_passes=False),
    out_type=jax.ShapeDtypeStruct((FIELDS * 4 * 128 * 8, 128), jnp.float32),
    scratch_types=[
        pltpu.VMEM((_IDX_PER_W,), jnp.int32),
        pltpu.VMEM((2, 128, 32), jnp.float32),
        pltpu.VMEM((2, 32, 128), jnp.float32),
        pltpu.SemaphoreType.DMA((2,)),
        pltpu.SemaphoreType.DMA((2,)),
    ],
)
def _k2(table_hbm, idx_hbm, out_hbm, idxv, rows, dest, gsem, osem):
    w = _wid()
    iota = _iota16()

    pltpu.sync_copy(idx_hbm.at[pl.ds(w * _IDX_PER_W, _IDX_PER_W)], idxv)

    def start_gather(i, s):
        pltpu.async_copy(table_hbm.at[idxv.at[pl.ds(128 * i, 128)]],
                         rows.at[s], gsem.at[s])

    def wait_gather(i, s):
        pltpu.make_async_copy(table_hbm.at[idxv.at[pl.ds(128 * i, 128)]],
                              rows.at[s], gsem.at[s]).wait()

    def out_bases(i):
        t = w * _COL_PER_W + i
        f = t // 128
        bb = t % 128
        return [4096 * f + 1024 * ii + 8 * bb for ii in range(4)]

    def start_outs(i, s):
        for ii, base in enumerate(out_bases(i)):
            pltpu.async_copy(dest.at[s].at[pl.ds(8 * ii, 8)],
                             out_hbm.at[pl.ds(base, 8)], osem.at[s])

    def wait_outs(i, s):
        for ii, base in enumerate(out_bases(i)):
            pltpu.make_async_copy(dest.at[s].at[pl.ds(8 * ii, 8)],
                                  out_hbm.at[pl.ds(base, 8)],
                                  osem.at[s]).wait()

    lvecs = [iota + 16 * q for q in range(8)]

    def transpose_col(s):
        def body(c0, cm):
            for q in range(8):
                v = plsc.load_gather(rows.at[s], [lvecs[q], cm])
                plsc.store_scatter(dest.at[s], [cm, lvecs[q]], v)
            return (cm + 1) & 31

        lax.fori_loop(0, 32, body, iota)

    start_gather(0, 0)
    start_gather(1, 1)

    def body_p(p, carry):
        for s in (0, 1):
            i = 2 * p + s
            wait_gather(i, s)

            @pl.when(i >= 2)
            def _():
                wait_outs(i - 2, s)

            transpose_col(s)
            start_outs(i, s)

            @pl.when(i + 2 < _COL_PER_W)
            def _():
                start_gather(i + 2, s)
        return carry

    lax.fori_loop(0, _COL_PER_W // 2, body_p, 0)

    for s in (0, 1):
        wait_outs(_COL_PER_W - 2 + s, s)


def kernel(input, weight):
    idx_t = input.T.reshape(-1)
    if idx_t.dtype != jnp.int32:
        idx_t = idx_t.astype(jnp.int32)
    wt = weight.T
    lin = _k1(wt)
    table = lin.reshape(VOCAB, EMBED)
    out_flat = _k2(table, idx_t)
    o = out_flat.reshape(FIELDS, 4, 128, 8, 128)
    o = o.transpose(2, 4, 0, 1, 3)
    return o.reshape(BATCH, FIELDS, EMBED)


# speedup vs baseline: 2.9816x; 1.0532x over previous
import functools

import jax
import jax.numpy as jnp
from jax import lax
from jax.experimental import pallas as pl
from jax.experimental.pallas import tpu as pltpu
from jax.experimental.pallas import tpu_sc as plsc

VOCAB = 1000000
EMBED = 32
BATCH = 16384
FIELDS = 26
_NC, _NS = 2, 16
_NW = _NC * _NS

_NBLK_FULL = VOCAB // 128
_TAIL = VOCAB - _NBLK_FULL * 128

_mesh = plsc.VectorSubcoreMesh(core_axis_name="c", subcore_axis_name="s")


def _wid():
    return lax.axis_index("s") * _NC + lax.axis_index("c")


def _iota16():
    return lax.iota(jnp.int32, 16)


@functools.partial(
    pl.kernel,
    mesh=_mesh,
    compiler_params=pltpu.CompilerParams(use_tc_tiling_on_sc=True,
                                         needs_layout_passes=False),
    out_type=jax.ShapeDtypeStruct((VOCAB // 4, 128), jnp.float32),
    scratch_types=[
        pltpu.VMEM((2, 32, 512), jnp.float32),
        pltpu.VMEM((2, 128, 128), jnp.float32),
        pltpu.VMEM((32, 64), jnp.float32),
        pltpu.VMEM((16, 128), jnp.float32),
        pltpu.SemaphoreType.DMA((2,)),
        pltpu.SemaphoreType.DMA((2,)),
    ],
)
def _k1(wt_hbm, lin_hbm, src, dst, tsrc, tdst, isem, osem):
    w = _wid()
    n4 = _NBLK_FULL // 4
    nw = (n4 - 1 - w) // 32 + 1
    iota = _iota16()

    def start_in(k, s):
        j = w + 32 * k
        pltpu.async_copy(wt_hbm.at[:, pl.ds(j * 512, 512)], src.at[s],
                         isem.at[s])

    def wait_in(k, s):
        j = w + 32 * k
        pltpu.make_async_copy(wt_hbm.at[:, pl.ds(j * 512, 512)], src.at[s],
                              isem.at[s]).wait()

    def start_out(k, s):
        j = w + 32 * k
        pltpu.async_copy(dst.at[s], lin_hbm.at[pl.ds(128 * j, 128)],
                         osem.at[s])

    def wait_out(k, s):
        j = w + 32 * k
        pltpu.make_async_copy(dst.at[s], lin_hbm.at[pl.ds(128 * j, 128)],
                              osem.at[s]).wait()

    def transpose_block(src_ref, dst_ref, nl0):
        cols = [iota + 16 * q for q in range(nl0)]
        wrows = [lax.shift_right_logical(c, 2) for c in cols]
        cbases = [lax.shift_left(c & 3, 5) for c in cols]

        def body(e0, em):
            for q in range(nl0):
                v = plsc.load_gather(src_ref, [em, cols[q]])
                plsc.store_scatter(dst_ref, [wrows[q], cbases[q] + em], v)
            return (em + 1) & 31

        lax.fori_loop(0, 32, body, iota)

    start_in(0, 0)
    start_in(1, 1)

    def body_p(p, carry):
        for s in (0, 1):
            k = 2 * p + s

            @pl.when(k < nw)
            def _():
                wait_in(k, s)

                @pl.when(k >= 2)
                def _():
                    wait_out(k - 2, s)

                transpose_block(src.at[s], dst.at[s], 32)
                start_out(k, s)

                @pl.when(k + 2 < nw)
                def _():
                    start_in(k + 2, s)
        return carry

    lax.fori_loop(0, (62 + 1) // 2, body_p, 0)

    for s in (0, 1):
        k_last = nw - 1 - ((nw - 1 - s) % 2)
        wait_out(k_last, s)

    @pl.when(w == _NBLK_FULL % 32)
    def _():
        pltpu.sync_copy(wt_hbm.at[:, pl.ds(_NBLK_FULL * 128, _TAIL)], tsrc)
        transpose_block(tsrc, tdst, _TAIL // 16)
        pltpu.sync_copy(tdst, lin_hbm.at[pl.ds(_NBLK_FULL * 32, _TAIL // 4)])


_NCOL = FIELDS * (BATCH // 128)
_COL_PER_W = _NCOL // _NW
_IDX_PER_W = _COL_PER_W * 128


@functools.partial(
    pl.kernel,
    mesh=_mesh,
    compiler_params=pltpu.CompilerParams(use_tc_tiling_on_sc=False,
                                         needs_layout_passes=False),
    out_type=jax.ShapeDtypeStruct((FIELDS * 4 * 128 * 8, 128), jnp.float32),
    scratch_types=[
        pltpu.VMEM((_IDX_PER_W,), jnp.int32),
        pltpu.VMEM((2, 256, 32), jnp.float32),
        pltpu.VMEM((2, 64, 128), jnp.float32),
        pltpu.SemaphoreType.DMA((2,)),
        pltpu.SemaphoreType.DMA((2,)),
    ],
)
def _k2(table_hbm, idx_hbm, out_hbm, idxv, rows, dest, gsem, osem):
    w = _wid()
    iota = _iota16()
    npair = _COL_PER_W // 2

    pltpu.sync_copy(idx_hbm.at[pl.ds(w * _IDX_PER_W, _IDX_PER_W)], idxv)

    def start_gather(i, s):
        pltpu.async_copy(table_hbm.at[idxv.at[pl.ds(256 * i, 256)]],
                         rows.at[s], gsem.at[s])

    def wait_gather(i, s):
        pltpu.make_async_copy(table_hbm.at[idxv.at[pl.ds(256 * i, 256)]],
                              rows.at[s], gsem.at[s]).wait()

    def out_bases(i, h):
        t = w * _COL_PER_W + 2 * i + h
        f = t // 128
        bb = t % 128
        return [4096 * f + 1024 * ii + 8 * bb for ii in range(4)]

    def start_outs(i, s):
        for h in range(2):
            for ii, base in enumerate(out_bases(i, h)):
                pltpu.async_copy(dest.at[s].at[pl.ds(32 * h + 8 * ii, 8)],
                                 out_hbm.at[pl.ds(base, 8)], osem.at[s])

    def wait_outs(i, s):
        for h in range(2):
            for ii, base in enumerate(out_bases(i, h)):
                pltpu.make_async_copy(dest.at[s].at[pl.ds(32 * h + 8 * ii, 8)],
                                      out_hbm.at[pl.ds(base, 8)],
                                      osem.at[s]).wait()

    lvecs = [iota + 16 * q for q in range(8)]

    def transpose_col(s):
        def body(c0, cm):
            for h in range(2):
                for q in range(8):
                    v = plsc.load_gather(rows.at[s],
                                         [lvecs[q] + 128 * h, cm])
                    plsc.store_scatter(dest.at[s],
                                       [cm + 32 * h, lvecs[q]], v)
            return (cm + 1) & 31

        lax.fori_loop(0, 32, body, iota)

    start_gather(0, 0)
    start_gather(1, 1)

    def body_p(p, carry):
        for s in (0, 1):
            i = 2 * p + s
            wait_gather(i, s)

            @pl.when(i >= 2)
            def _():
                wait_outs(i - 2, s)

            transpose_col(s)
            start_outs(i, s)

            @pl.when(i + 2 < npair)
            def _():
                start_gather(i + 2, s)
        return carry

    lax.fori_loop(0, npair // 2, body_p, 0)

    for s in (0, 1):
        wait_outs(npair - 2 + s, s)


def kernel(input, weight):
    idx_t = input.T.reshape(-1)
    if idx_t.dtype != jnp.int32:
        idx_t = idx_t.astype(jnp.int32)
    wt = weight.T
    lin = _k1(wt)
    table = lin.reshape(VOCAB, EMBED)
    out_flat = _k2(table, idx_t)
    o = out_flat.reshape(FIELDS, 4, 128, 8, 128)
    o = o.transpose(2, 4, 0, 1, 3)
    return o.reshape(BATCH, FIELDS, EMBED)


# speedup vs baseline: 2.9962x; 1.0049x over previous
import functools

import jax
import jax.numpy as jnp
from jax import lax
from jax.experimental import pallas as pl
from jax.experimental.pallas import tpu as pltpu
from jax.experimental.pallas import tpu_sc as plsc

VOCAB = 1000000
EMBED = 32
BATCH = 16384
FIELDS = 26
_NC, _NS = 2, 16
_NW = _NC * _NS

_NBLK_FULL = VOCAB // 128
_TAIL = VOCAB - _NBLK_FULL * 128

_mesh = plsc.VectorSubcoreMesh(core_axis_name="c", subcore_axis_name="s")


def _wid():
    return lax.axis_index("s") * _NC + lax.axis_index("c")


def _iota16():
    return lax.iota(jnp.int32, 16)


@functools.partial(
    pl.kernel,
    mesh=_mesh,
    compiler_params=pltpu.CompilerParams(use_tc_tiling_on_sc=True,
                                         needs_layout_passes=False),
    out_type=jax.ShapeDtypeStruct((VOCAB // 4, 128), jnp.float32),
    scratch_types=[
        pltpu.VMEM((2, 32, 512), jnp.float32),
        pltpu.VMEM((2, 128, 128), jnp.float32),
        pltpu.VMEM((32, 64), jnp.float32),
        pltpu.VMEM((16, 128), jnp.float32),
        pltpu.SemaphoreType.DMA((2,)),
        pltpu.SemaphoreType.DMA((2,)),
    ],
)
def _k1(wt_hbm, lin_hbm, src, dst, tsrc, tdst, isem, osem):
    w = _wid()
    n4 = _NBLK_FULL // 4
    nw = (n4 - 1 - w) // 32 + 1
    iota = _iota16()

    def start_in(k, s):
        j = w + 32 * k
        pltpu.async_copy(wt_hbm.at[:, pl.ds(j * 512, 512)], src.at[s],
                         isem.at[s])

    def wait_in(k, s):
        j = w + 32 * k
        pltpu.make_async_copy(wt_hbm.at[:, pl.ds(j * 512, 512)], src.at[s],
                              isem.at[s]).wait()

    def start_out(k, s):
        j = w + 32 * k
        pltpu.async_copy(dst.at[s], lin_hbm.at[pl.ds(128 * j, 128)],
                         osem.at[s])

    def wait_out(k, s):
        j = w + 32 * k
        pltpu.make_async_copy(dst.at[s], lin_hbm.at[pl.ds(128 * j, 128)],
                              osem.at[s]).wait()

    def transpose_block(src_ref, dst_ref, nl0):
        cols = [iota + 16 * q for q in range(nl0)]
        wrows = [lax.shift_right_logical(c, 2) for c in cols]
        cbases = [lax.shift_left(c & 3, 5) for c in cols]

        def body(e0, carry):
            em0, em1 = carry
            for q in range(nl0):
                v0 = plsc.load_gather(src_ref, [em0, cols[q]])
                plsc.store_scatter(dst_ref, [wrows[q], cbases[q] + em0], v0)
                v1 = plsc.load_gather(src_ref, [em1, cols[q]])
                plsc.store_scatter(dst_ref, [wrows[q], cbases[q] + em1], v1)
            return ((em0 + 2) & 31, (em1 + 2) & 31)

        lax.fori_loop(0, 16, body, (iota, (iota + 1) & 31))

    start_in(0, 0)
    start_in(1, 1)

    def body_p(p, carry):
        for s in (0, 1):
            k = 2 * p + s

            @pl.when(k < nw)
            def _():
                wait_in(k, s)

                @pl.when(k >= 2)
                def _():
                    wait_out(k - 2, s)

                transpose_block(src.at[s], dst.at[s], 32)
                start_out(k, s)

                @pl.when(k + 2 < nw)
                def _():
                    start_in(k + 2, s)
        return carry

    lax.fori_loop(0, (62 + 1) // 2, body_p, 0)

    for s in (0, 1):
        k_last = nw - 1 - ((nw - 1 - s) % 2)
        wait_out(k_last, s)

    @pl.when(w == _NBLK_FULL % 32)
    def _():
        pltpu.sync_copy(wt_hbm.at[:, pl.ds(_NBLK_FULL * 128, _TAIL)], tsrc)
        transpose_block(tsrc, tdst, _TAIL // 16)
        pltpu.sync_copy(tdst, lin_hbm.at[pl.ds(_NBLK_FULL * 32, _TAIL // 4)])


_NCOL = FIELDS * (BATCH // 128)
_COL_PER_W = _NCOL // _NW
_IDX_PER_W = _COL_PER_W * 128


@functools.partial(
    pl.kernel,
    mesh=_mesh,
    compiler_params=pltpu.CompilerParams(use_tc_tiling_on_sc=False,
                                         needs_layout_passes=False),
    out_type=jax.ShapeDtypeStruct((FIELDS * 4 * 128 * 8, 128), jnp.float32),
    scratch_types=[
        pltpu.VMEM((_IDX_PER_W,), jnp.int32),
        pltpu.VMEM((2, 256, 32), jnp.float32),
        pltpu.VMEM((2, 64, 128), jnp.float32),
        pltpu.SemaphoreType.DMA((2,)),
        pltpu.SemaphoreType.DMA((2,)),
    ],
)
def _k2(table_hbm, idx_hbm, out_hbm, idxv, rows, dest, gsem, osem):
    w = _wid()
    iota = _iota16()
    npair = _COL_PER_W // 2

    pltpu.sync_copy(idx_hbm.at[pl.ds(w * _IDX_PER_W, _IDX_PER_W)], idxv)

    def start_gather(i, s):
        pltpu.async_copy(table_hbm.at[idxv.at[pl.ds(256 * i, 256)]],
                         rows.at[s], gsem.at[s])

    def wait_gather(i, s):
        pltpu.make_async_copy(table_hbm.at[idxv.at[pl.ds(256 * i, 256)]],
                              rows.at[s], gsem.at[s]).wait()

    def out_bases(i, h):
        t = w * _COL_PER_W + 2 * i + h
        f = t // 128
        bb = t % 128
        return [4096 * f + 1024 * ii + 8 * bb for ii in range(4)]

    def start_outs(i, s):
        for h in range(2):
            for ii, base in enumerate(out_bases(i, h)):
                pltpu.async_copy(dest.at[s].at[pl.ds(32 * h + 8 * ii, 8)],
                                 out_hbm.at[pl.ds(base, 8)], osem.at[s])

    def wait_outs(i, s):
        for h in range(2):
            for ii, base in enumerate(out_bases(i, h)):
                pltpu.make_async_copy(dest.at[s].at[pl.ds(32 * h + 8 * ii, 8)],
                                      out_hbm.at[pl.ds(base, 8)],
                                      osem.at[s]).wait()

    lvecs = [iota + 16 * q for q in range(8)]
    lvecs2 = [[iota + 16 * q + 128 * h for q in range(8)] for h in range(2)]

    def transpose_col(s):
        def body(c0, carry):
            cm0, cm1 = carry
            for h in range(2):
                for q in range(8):
                    lv = lvecs2[h][q]
                    v0 = plsc.load_gather(rows.at[s], [lv, cm0])
                    plsc.store_scatter(dest.at[s], [cm0 + 32 * h, lvecs[q]],
                                       v0)
                    v1 = plsc.load_gather(rows.at[s], [lv, cm1])
                    plsc.store_scatter(dest.at[s], [cm1 + 32 * h, lvecs[q]],
                                       v1)
            return ((cm0 + 2) & 31, (cm1 + 2) & 31)

        lax.fori_loop(0, 16, body, (iota, (iota + 1) & 31))

    start_gather(0, 0)
    start_gather(1, 1)

    def body_p(p, carry):
        for s in (0, 1):
            i = 2 * p + s
            wait_gather(i, s)

            @pl.when(i >= 2)
            def _():
                wait_outs(i - 2, s)

            transpose_col(s)
            start_outs(i, s)

            @pl.when(i + 2 < npair)
            def _():
                start_gather(i + 2, s)
        return carry

    lax.fori_loop(0, npair // 2, body_p, 0)

    for s in (0, 1):
        wait_outs(npair - 2 + s, s)


def kernel(input, weight):
    idx_t = input.T.reshape(-1)
    if idx_t.dtype != jnp.int32:
        idx_t = idx_t.astype(jnp.int32)
    wt = weight.T
    lin = _k1(wt)
    table = lin.reshape(VOCAB, EMBED)
    out_flat = _k2(table, idx_t)
    o = out_flat.reshape(FIELDS, 4, 128, 8, 128)
    o = o.transpose(2, 4, 0, 1, 3)
    return o.reshape(BATCH, FIELDS, EMBED)
